# Initial kernel scaffold; baseline (speedup 1.0000x reference)
import jax
import jax.numpy as jnp
from jax.experimental import pallas as pl


def kernel(user_id, item_id, edge_index, edge_time, user_index, last_item_index, user_emb, item_emb, user_w, item_w, user_upd, item_upd, i_freq, i_phase, i_freq_k, i_phase_k, u_freq, u_phase, u_freq_k, u_phase_k, unified_w, dense_w, dense_b):
    raise NotImplementedError("write your pallas kernel here")


# speedup vs baseline: 3.9970x; 3.9970x over previous
import functools

import jax
import jax.numpy as jnp
from jax import lax
from jax.experimental import pallas as pl
from jax.experimental.pallas import tpu as pltpu
from jax.experimental.pallas import tpu_sc as plsc

N_USERS = 10000
N_ITEMS = 10000
E = 320000
D = 128
L = 3
B = 100
W136 = 136
N_ACC = 10240

_MESH = dict(core_axis_name="c", subcore_axis_name="s")
NC, NS = 2, 16


def _gather_pair(table_a, idx_a, table_b, idx_b, n_out, chunk):
    per_tile = n_out // (NC * NS)
    iters = per_tile // chunk

    @functools.partial(
        pl.kernel,
        mesh=plsc.VectorSubcoreMesh(num_cores=NC, **_MESH),
        out_type=[
            jax.ShapeDtypeStruct((n_out, D), jnp.float32),
            jax.ShapeDtypeStruct((n_out, D), jnp.float32),
        ],
        scratch_types=[
            pltpu.VMEM((chunk,), jnp.int32),
            pltpu.VMEM((chunk, D), jnp.float32),
            pltpu.SemaphoreType.DMA,
        ],
    )
    def k(ta, ia, tb, ib, oa, ob, idx_v, rows_v, sem):
        wid = lax.axis_index("s") * NC + lax.axis_index("c")
        for tab, idx, out in ((ta, ia, oa), (tb, ib, ob)):
            def body(i, carry, tab=tab, idx=idx, out=out):
                base = wid * per_tile + i * chunk
                pltpu.sync_copy(idx.at[pl.ds(base, chunk)], idx_v)
                pltpu.async_copy(tab.at[idx_v], rows_v, sem).wait()
                pltpu.sync_copy(rows_v, out.at[pl.ds(base, chunk)])
                return carry
            lax.fori_loop(0, iters, body, 0)

    return k(table_a, idx_a, table_b, idx_b)


def _scatter_add(vals, idx, zrows):
    chunk = 200
    per_tile = E // (NC * NS)
    iters = per_tile // chunk
    zrows_n = N_ACC // NS

    @functools.partial(
        pl.kernel,
        mesh=plsc.VectorSubcoreMesh(num_cores=NC, **_MESH),
        out_type=jax.ShapeDtypeStruct((NC, N_ACC, D), jnp.float32),
        scratch_types=[
            pltpu.VMEM_SHARED((N_ACC, D), jnp.float32),
            pltpu.VMEM((chunk,), jnp.int32),
            pltpu.VMEM((chunk, D), jnp.float32),
        ],
    )
    def k(v, ix, zr, out, acc, idx_v, vals_v):
        cid = lax.axis_index("c")
        sid = lax.axis_index("s")
        wid = sid * NC + cid
        pltpu.sync_copy(zr, acc.at[pl.ds(sid * zrows_n, zrows_n)])
        plsc.subcore_barrier()

        def body(i, carry):
            base = wid * per_tile + i * chunk
            pltpu.sync_copy(ix.at[pl.ds(base, chunk)], idx_v)
            pltpu.sync_copy(v.at[pl.ds(base, chunk)], vals_v)
            pltpu.sync_copy(vals_v, acc.at[idx_v], add=True)
            return carry
        lax.fori_loop(0, iters, body, 0)
        plsc.subcore_barrier()
        pltpu.sync_copy(acc.at[pl.ds(sid * zrows_n, zrows_n)],
                        out.at[cid, pl.ds(sid * zrows_n, zrows_n)])

    return k(vals, idx, zrows)


def _gather_final(t0, t1, t2, t3, i0, i1, i2, i3):
    nb = 256
    per_tile = nb // (NC * NS)

    @functools.partial(
        pl.kernel,
        mesh=plsc.VectorSubcoreMesh(num_cores=NC, **_MESH),
        out_type=[jax.ShapeDtypeStruct((nb, D), jnp.float32) for _ in range(4)],
        scratch_types=[
            pltpu.VMEM((per_tile,), jnp.int32),
            pltpu.VMEM((per_tile, D), jnp.float32),
            pltpu.SemaphoreType.DMA,
        ],
    )
    def k(a, b, c, d, ia, ib, ic, id_, oa, ob, oc, od, idx_v, rows_v, sem):
        wid = lax.axis_index("s") * NC + lax.axis_index("c")
        base = wid * per_tile
        for tab, idx, out in ((a, ia, oa), (b, ib, ob), (c, ic, oc),
                              (d, id_, od)):
            pltpu.sync_copy(idx.at[pl.ds(base, per_tile)], idx_v)
            pltpu.async_copy(tab.at[idx_v], rows_v, sem).wait()
            pltpu.sync_copy(rows_v, out.at[pl.ds(base, per_tile)])

    return k(t0, t1, t2, t3, i0, i1, i2, i3)


def _mm(x, w):
    n = x.shape[0]
    blk = 2000

    def body(x_ref, w_ref, o_ref):
        o_ref[...] = jnp.dot(x_ref[...], w_ref[...],
                             preferred_element_type=jnp.float32)

    return pl.pallas_call(
        body,
        grid=(n // blk,),
        in_specs=[
            pl.BlockSpec((blk, D), lambda i: (i, 0)),
            pl.BlockSpec((D, D), lambda i: (0, 0)),
        ],
        out_specs=pl.BlockSpec((blk, D), lambda i: (i, 0)),
        out_shape=jax.ShapeDtypeStruct((n, D), jnp.float32),
    )(x, w)


def _edge_kernel(t_col, uh_e, ih_e, fp):
    blk = 2000
    inv_scale = float(1.0 / (D ** 0.5))

    def body(t_ref, u_ref, i_ref, fp_ref, vi_ref, si_ref, vu_ref, su_ref):
        t = t_ref[...]
        uh = u_ref[...]
        ih = i_ref[...]
        fp_ = fp_ref[...]
        z = jnp.zeros((blk, D - 1), jnp.float32)

        te_i = jnp.cos(t * fp_[0:1, :] + fp_[1:2, :])
        s_i = jnp.exp(jnp.sum((te_i + uh) * ih, axis=1, keepdims=True)
                      * inv_scale)
        tek = jnp.cos(t * fp_[2:3, :] + fp_[3:4, :])
        vi_ref[...] = s_i * (uh + tek)
        si_ref[...] = jnp.concatenate([s_i, z], axis=1)

        te_u = jnp.cos(t * fp_[4:5, :] + fp_[5:6, :])
        s_u = jnp.exp(jnp.sum((te_u + ih) * uh, axis=1, keepdims=True)
                      * inv_scale)
        tuk = jnp.cos(t * fp_[6:7, :] + fp_[7:8, :])
        vu_ref[...] = s_u * (ih + tuk)
        su_ref[...] = jnp.concatenate([s_u, z], axis=1)

    return pl.pallas_call(
        body,
        grid=(E // blk,),
        in_specs=[
            pl.BlockSpec((blk, 1), lambda i: (i, 0)),
            pl.BlockSpec((blk, D), lambda i: (i, 0)),
            pl.BlockSpec((blk, D), lambda i: (i, 0)),
            pl.BlockSpec((8, D), lambda i: (0, 0)),
        ],
        out_specs=[pl.BlockSpec((blk, D), lambda i: (i, 0))
                   for _ in range(4)],
        out_shape=[jax.ShapeDtypeStruct((E, D), jnp.float32)
                   for _ in range(4)],
    )(t_col, uh_e, ih_e, fp)


def _update(acc_v, acc_s, h_old, upd):
    n = h_old.shape[0]
    blk = 2000

    def body(v_ref, s_ref, h_ref, w_ref, o_ref):
        num = v_ref[0] + v_ref[1]
        den = s_ref[0][:, 0:1] + s_ref[1][:, 0:1]
        h_new = num / (den + 1e-9)
        y = (jnp.dot(h_new, w_ref[:D, :], preferred_element_type=jnp.float32)
             + jnp.dot(h_ref[...], w_ref[D:, :],
                       preferred_element_type=jnp.float32))
        o_ref[...] = jnp.where(y > 0, y, jnp.exp(jnp.minimum(y, 0.0)) - 1.0)

    return pl.pallas_call(
        body,
        grid=(n // blk,),
        in_specs=[
            pl.BlockSpec((NC, blk, D), lambda i: (0, i, 0)),
            pl.BlockSpec((NC, blk, D), lambda i: (0, i, 0)),
            pl.BlockSpec((blk, D), lambda i: (i, 0)),
            pl.BlockSpec((2 * D, D), lambda i: (0, 0)),
        ],
        out_specs=pl.BlockSpec((blk, D), lambda i: (i, 0)),
        out_shape=jax.ShapeDtypeStruct((n, D), jnp.float32),
    )(acc_v, acc_s, h_old, upd)


def _head(g0, g1, g2, g3, unified_w, dense_w, dense_b):
    def body(a_ref, b_ref, c_ref, d_ref, uw_ref, dw_ref, db_ref, o_ref):
        cat = jnp.concatenate(
            [a_ref[...], b_ref[...], c_ref[...], d_ref[...]], axis=1)
        u = jnp.dot(cat, uw_ref[...], preferred_element_type=jnp.float32)
        y = jnp.dot(u, dw_ref[...],
                    preferred_element_type=jnp.float32) + db_ref[...]
        o_ref[...] = jnp.maximum(y, 0.0)

    return pl.pallas_call(
        body,
        out_shape=jax.ShapeDtypeStruct((256, 3), jnp.float32),
    )(g0, g1, g2, g3, unified_w, dense_w, jnp.reshape(dense_b, (1, 3)))


def kernel(user_id, item_id, edge_index, edge_time, user_index,
           last_item_index, user_emb, item_emb, user_w, item_w, user_upd,
           item_upd, i_freq, i_phase, i_freq_k, i_phase_k, u_freq, u_phase,
           u_freq_k, u_phase_k, unified_w, dense_w, dense_b):
    src = edge_index[0].astype(jnp.int32)
    dst = edge_index[1].astype(jnp.int32)
    t_col = edge_time.reshape(E, 1)
    zrows = jnp.zeros((N_ACC // NS, D), jnp.float32)

    pad_n = 10240
    uid = jnp.concatenate(
        [user_id.astype(jnp.int32),
         jnp.zeros((pad_n - N_USERS,), jnp.int32)])
    iid = jnp.concatenate(
        [item_id.astype(jnp.int32),
         jnp.zeros((pad_n - N_ITEMS,), jnp.int32)])
    user_p, item_p = _gather_pair(user_emb, uid, item_emb, iid, pad_n, 320)
    user_ = user_p[:N_USERS]
    item_ = item_p[:N_ITEMS]

    pad_b = jnp.zeros((256 - B,), jnp.int32)
    fu = jnp.concatenate(
        [(jnp.arange(B, dtype=jnp.int32) * (N_USERS // B)
          + user_index.astype(jnp.int32)), pad_b])
    fi = jnp.concatenate(
        [(jnp.arange(B, dtype=jnp.int32) * (N_ITEMS // B)
          + last_item_index.astype(jnp.int32)), pad_b])

    user_layers = []
    for l in range(L):
        uh = _mm(user_, user_w[l])
        ih = _mm(item_, item_w[l])
        uh_e, ih_e = _gather_pair(uh, src, ih, dst, E, 400)
        fp = jnp.stack([i_freq[l], i_phase[l], i_freq_k[l], i_phase_k[l],
                        u_freq[l], u_phase[l], u_freq_k[l], u_phase_k[l]])
        v_i, sv_i, v_u, sv_u = _edge_kernel(t_col, uh_e, ih_e, fp)
        acc_i = _scatter_add(v_i, dst, zrows)
        acc_si = _scatter_add(sv_i, dst, zrows)
        acc_u = _scatter_add(v_u, src, zrows)
        acc_su = _scatter_add(sv_u, src, zrows)
        user_ = _update(acc_u, acc_su, user_, user_upd[l])
        item_ = _update(acc_i, acc_si, item_, item_upd[l])
        user_layers.append(user_)

    g0, g1, g2, g3 = _gather_final(
        user_layers[0], user_layers[1], user_layers[2], item_,
        fu, fu, fu, fi)
    out = _head(g0, g1, g2, g3, unified_w, dense_w, dense_b)
    return out[:B]
